# Initial kernel scaffold; baseline (speedup 1.0000x reference)
import jax
import jax.numpy as jnp
from jax.experimental import pallas as pl


def kernel(x, edge_index, Wl1, bl1, Wr1, Wl2, bl2, Wr2):
    raise NotImplementedError("write your pallas kernel here")


# speedup vs baseline: 7.2886x; 7.2886x over previous
import functools

import jax
import jax.numpy as jnp
from jax import lax
from jax.experimental import pallas as pl
from jax.experimental.pallas import tpu as pltpu
from jax.experimental.pallas import tpu_sc as plsc

N = 10000
E = 320000
IN_DIM = 128
CODE = 20
K = 10
D = 32
EPS = 1e-20

NC = 2
NS = 16
E_PER_SC = E // NC
E_PER_TILE = E_PER_SC // NS
CHUNK = 80
NCH = E_PER_TILE // CHUNK
ROWS_A = 624
ROWS_LAST = N - 15 * ROWS_A

_f32 = jnp.float32


def _proj_body(x_ref, wl_ref, wr_ref, p_ref, r_ref):
    xb = x_ref[...]
    p = jnp.dot(xb, wl_ref[...], preferred_element_type=_f32)
    col = lax.broadcasted_iota(jnp.int32, p.shape, 1)
    p_ref[...] = p + jnp.where(col == CODE, _f32(1.0), _f32(0.0))
    r_ref[...] = jnp.dot(xb, wr_ref[...], preferred_element_type=_f32)


def _proj(x, wl_pad_t, wr_t):
    blk = 1000
    return pl.pallas_call(
        _proj_body,
        grid=(N // blk,),
        in_specs=[
            pl.BlockSpec((blk, IN_DIM), lambda i: (i, 0)),
            pl.BlockSpec((IN_DIM, D), lambda i: (0, 0)),
            pl.BlockSpec((IN_DIM, CODE), lambda i: (0, 0)),
        ],
        out_specs=[
            pl.BlockSpec((blk, D), lambda i: (i, 0)),
            pl.BlockSpec((blk, CODE), lambda i: (i, 0)),
        ],
        out_shape=[
            jax.ShapeDtypeStruct((N, D), _f32),
            jax.ShapeDtypeStruct((N, CODE), _f32),
        ],
    )(x, wl_pad_t, wr_t)


def _segsum_body(vals_hbm, src_hbm, dst_hbm, zeros_hbm, out_hbm,
                 src_v, dst_v, rows_v, acc_sh, sem):
    cid = lax.axis_index("c")
    sid = lax.axis_index("s")
    row0 = pl.multiple_of(sid * ROWS_A, 8)

    @pl.when(sid < NS - 1)
    def _():
        pltpu.sync_copy(zeros_hbm.at[pl.ds(row0, ROWS_A)],
                        acc_sh.at[pl.ds(row0, ROWS_A)])

    @pl.when(sid == NS - 1)
    def _():
        pltpu.sync_copy(zeros_hbm.at[pl.ds((NS - 1) * ROWS_A, ROWS_LAST)],
                        acc_sh.at[pl.ds((NS - 1) * ROWS_A, ROWS_LAST)])

    plsc.subcore_barrier()

    base = cid * E_PER_SC + sid * E_PER_TILE

    def body(j, carry):
        off = base + j * CHUNK
        pltpu.sync_copy(src_hbm.at[pl.ds(off, CHUNK)], src_v)
        pltpu.sync_copy(dst_hbm.at[pl.ds(off, CHUNK)], dst_v)
        pltpu.async_copy(vals_hbm.at[src_v], rows_v, sem).wait()
        pltpu.sync_copy(rows_v, acc_sh.at[dst_v], add=True)
        return carry

    lax.fori_loop(0, NCH, body, 0)
    plsc.subcore_barrier()

    @pl.when(sid < NS - 1)
    def _():
        pltpu.sync_copy(acc_sh.at[pl.ds(row0, ROWS_A)],
                        out_hbm.at[cid, pl.ds(row0, ROWS_A)])

    @pl.when(sid == NS - 1)
    def _():
        pltpu.sync_copy(acc_sh.at[pl.ds((NS - 1) * ROWS_A, ROWS_LAST)],
                        out_hbm.at[cid, pl.ds((NS - 1) * ROWS_A, ROWS_LAST)])


@functools.lru_cache(maxsize=1)
def _get_segsum():
    mesh = plsc.VectorSubcoreMesh(
        core_axis_name="c", subcore_axis_name="s",
        num_cores=NC, num_subcores=NS)
    return pl.kernel(
        _segsum_body,
        out_type=jax.ShapeDtypeStruct((NC, N, D), _f32),
        mesh=mesh,
        compiler_params=pltpu.CompilerParams(use_tc_tiling_on_sc=False),
        scratch_types=[
            pltpu.VMEM((CHUNK,), jnp.int32),
            pltpu.VMEM((CHUNK,), jnp.int32),
            pltpu.VMEM((CHUNK, D), _f32),
            pltpu.VMEM_SHARED((N, D), _f32),
            pltpu.SemaphoreType.DMA,
        ],
    )


def _segsum(vals, src, dst, zeros):
    return _get_segsum()(vals, src, dst, zeros)


def _code_body(acc_ref, r_ref, bl_ref, u_ref, y_ref):
    acc = acc_ref[0] + acc_ref[1]
    deg = acc[:, CODE:CODE + 1]
    invdeg = _f32(1.0) / jnp.maximum(deg, _f32(1.0))
    h = acc[:, :CODE] * invdeg + bl_ref[...] + r_ref[...]
    g = -jnp.log(-jnp.log(u_ref[...] + _f32(EPS)))
    z = h + g
    z0 = z[:, :K]
    z1 = z[:, K:]
    e0 = jnp.exp(z0 - jnp.max(z0, axis=1, keepdims=True))
    e1 = jnp.exp(z1 - jnp.max(z1, axis=1, keepdims=True))
    y0 = e0 / jnp.sum(e0, axis=1, keepdims=True)
    y1 = e1 / jnp.sum(e1, axis=1, keepdims=True)
    pad = jnp.zeros((z.shape[0], D - CODE - 1), _f32)
    y_ref[...] = jnp.concatenate([y0, y1, invdeg, pad], axis=1)


def _code(acc1, r, bl1, u):
    blk = 1000
    return pl.pallas_call(
        _code_body,
        grid=(N // blk,),
        in_specs=[
            pl.BlockSpec((NC, blk, D), lambda i: (0, i, 0)),
            pl.BlockSpec((blk, CODE), lambda i: (i, 0)),
            pl.BlockSpec((1, CODE), lambda i: (0, 0)),
            pl.BlockSpec((blk, CODE), lambda i: (i, 0)),
        ],
        out_specs=pl.BlockSpec((blk, D), lambda i: (i, 0)),
        out_shape=jax.ShapeDtypeStruct((N, D), _f32),
    )(acc1, r, bl1, u)


def _dec_body(acc_ref, y_ref, wl_ref, wr_ref, bl_ref, out_ref):
    acc = acc_ref[0] + acc_ref[1]
    ypad = y_ref[...]
    invdeg = ypad[:, CODE:CODE + 1]
    mean2 = acc[:, :CODE] * invdeg
    out_ref[...] = (jnp.dot(mean2, wl_ref[...], preferred_element_type=_f32)
                    + jnp.dot(ypad[:, :CODE], wr_ref[...],
                              preferred_element_type=_f32)
                    + bl_ref[...])


def _dec(acc2, y_pad, wl2_t, wr2_t, bl2):
    blk = 1000
    return pl.pallas_call(
        _dec_body,
        grid=(N // blk,),
        in_specs=[
            pl.BlockSpec((NC, blk, D), lambda i: (0, i, 0)),
            pl.BlockSpec((blk, D), lambda i: (i, 0)),
            pl.BlockSpec((CODE, IN_DIM), lambda i: (0, 0)),
            pl.BlockSpec((CODE, IN_DIM), lambda i: (0, 0)),
            pl.BlockSpec((1, IN_DIM), lambda i: (0, 0)),
        ],
        out_specs=pl.BlockSpec((blk, IN_DIM), lambda i: (i, 0)),
        out_shape=jax.ShapeDtypeStruct((N, IN_DIM), _f32),
    )(acc2, y_pad, wl2_t, wr2_t, bl2)


def kernel(x, edge_index, Wl1, bl1, Wr1, Wl2, bl2, Wr2):
    src = edge_index[0].astype(jnp.int32)
    dst = edge_index[1].astype(jnp.int32)

    wl_pad_t = jnp.zeros((IN_DIM, D), _f32).at[:, :CODE].set(Wl1.T)
    p_pad, r = _proj(x, wl_pad_t, Wr1.T)

    zeros = jnp.zeros((N, D), _f32)
    acc1 = _segsum(p_pad, src, dst, zeros)

    u = jax.random.uniform(jax.random.key(123), (N, 2, K),
                           dtype=_f32).reshape(N, CODE)
    y_pad = _code(acc1, r, bl1.reshape(1, CODE), u)

    acc2 = _segsum(y_pad, src, dst, zeros)

    return _dec(acc2, y_pad, Wl2.T, Wr2.T, bl2.reshape(1, IN_DIM))


# speedup vs baseline: 13.3183x; 1.8273x over previous
import functools

import jax
import jax.numpy as jnp
from jax import lax
from jax.experimental import pallas as pl
from jax.experimental.pallas import tpu as pltpu
from jax.experimental.pallas import tpu_sc as plsc

N = 10000
E = 320000
IN_DIM = 128
CODE = 20
K = 10
D = 32
EPS = 1e-20

NC = 2
NS = 16
E_PER_SC = E // NC
E_PER_TILE = E_PER_SC // NS
CHUNK = 80
NCH = E_PER_TILE // CHUNK
ROWS_A = 624
ROWS_LAST = N - 15 * ROWS_A

_f32 = jnp.float32


def _proj_body(x_ref, wl_ref, wr_ref, p_ref, r_ref):
    xb = x_ref[...]
    p = jnp.dot(xb, wl_ref[...], preferred_element_type=_f32)
    col = lax.broadcasted_iota(jnp.int32, p.shape, 1)
    p_ref[...] = p + jnp.where(col == CODE, _f32(1.0), _f32(0.0))
    r_ref[...] = jnp.dot(xb, wr_ref[...], preferred_element_type=_f32)


def _proj(x, wl_pad_t, wr_t):
    blk = 1000
    return pl.pallas_call(
        _proj_body,
        grid=(N // blk,),
        in_specs=[
            pl.BlockSpec((blk, IN_DIM), lambda i: (i, 0)),
            pl.BlockSpec((IN_DIM, D), lambda i: (0, 0)),
            pl.BlockSpec((IN_DIM, CODE), lambda i: (0, 0)),
        ],
        out_specs=[
            pl.BlockSpec((blk, D), lambda i: (i, 0)),
            pl.BlockSpec((blk, CODE), lambda i: (i, 0)),
        ],
        out_shape=[
            jax.ShapeDtypeStruct((N, D), _f32),
            jax.ShapeDtypeStruct((N, CODE), _f32),
        ],
    )(x, wl_pad_t, wr_t)


def _segsum_body(vals_hbm, src_hbm, dst_hbm, zeros_hbm, out_hbm,
                 srcs_v, dsts_v, rows_v, acc_sh, gsem, ssem):
    cid = lax.axis_index("c")
    sid = lax.axis_index("s")
    tid = cid * NS + sid
    row0 = pl.multiple_of(sid * ROWS_A, 8)

    @pl.when(sid < NS - 1)
    def _():
        pltpu.sync_copy(zeros_hbm.at[pl.ds(row0, ROWS_A)],
                        acc_sh.at[pl.ds(row0, ROWS_A)])

    @pl.when(sid == NS - 1)
    def _():
        pltpu.sync_copy(zeros_hbm.at[pl.ds((NS - 1) * ROWS_A, ROWS_LAST)],
                        acc_sh.at[pl.ds((NS - 1) * ROWS_A, ROWS_LAST)])

    pltpu.sync_copy(src_hbm.at[tid], srcs_v)
    pltpu.sync_copy(dst_hbm.at[tid], dsts_v)
    plsc.subcore_barrier()

    pltpu.async_copy(vals_hbm.at[srcs_v.at[0]], rows_v.at[0], gsem)

    def body(j, carry):
        b = lax.rem(j, 2)
        pltpu.make_async_copy(vals_hbm.at[srcs_v.at[j]], rows_v.at[b],
                              gsem).wait()
        pltpu.async_copy(rows_v.at[b], acc_sh.at[dsts_v.at[j]], ssem.at[b],
                         add=True)

        @pl.when(j > 0)
        def _():
            pltpu.make_async_copy(rows_v.at[1 - b],
                                  acc_sh.at[dsts_v.at[j - 1]],
                                  ssem.at[1 - b]).wait()

        @pl.when(j < NCH - 1)
        def _():
            pltpu.async_copy(vals_hbm.at[srcs_v.at[j + 1]], rows_v.at[1 - b],
                             gsem)

        return carry

    lax.fori_loop(0, NCH, body, 0)
    last = (NCH - 1) % 2
    pltpu.make_async_copy(rows_v.at[last], acc_sh.at[dsts_v.at[NCH - 1]],
                          ssem.at[last]).wait()
    plsc.subcore_barrier()

    @pl.when(sid < NS - 1)
    def _():
        pltpu.sync_copy(acc_sh.at[pl.ds(row0, ROWS_A)],
                        out_hbm.at[cid, pl.ds(row0, ROWS_A)])

    @pl.when(sid == NS - 1)
    def _():
        pltpu.sync_copy(acc_sh.at[pl.ds((NS - 1) * ROWS_A, ROWS_LAST)],
                        out_hbm.at[cid, pl.ds((NS - 1) * ROWS_A, ROWS_LAST)])


@functools.lru_cache(maxsize=1)
def _get_segsum():
    mesh = plsc.VectorSubcoreMesh(
        core_axis_name="c", subcore_axis_name="s",
        num_cores=NC, num_subcores=NS)
    return pl.kernel(
        _segsum_body,
        out_type=jax.ShapeDtypeStruct((NC, N, D), _f32),
        mesh=mesh,
        compiler_params=pltpu.CompilerParams(use_tc_tiling_on_sc=False),
        scratch_types=[
            pltpu.VMEM((NCH, CHUNK), jnp.int32),
            pltpu.VMEM((NCH, CHUNK), jnp.int32),
            pltpu.VMEM((2, CHUNK, D), _f32),
            pltpu.VMEM_SHARED((N, D), _f32),
            pltpu.SemaphoreType.DMA,
            pltpu.SemaphoreType.DMA((2,)),
        ],
    )


def _segsum(vals, src3, dst3, zeros):
    return _get_segsum()(vals, src3, dst3, zeros)


def _code_body(acc_ref, r_ref, bl_ref, u_ref, y_ref):
    acc = acc_ref[0] + acc_ref[1]
    deg = acc[:, CODE:CODE + 1]
    invdeg = _f32(1.0) / jnp.maximum(deg, _f32(1.0))
    h = acc[:, :CODE] * invdeg + bl_ref[...] + r_ref[...]
    g = -jnp.log(-jnp.log(u_ref[...] + _f32(EPS)))
    z = h + g
    z0 = z[:, :K]
    z1 = z[:, K:]
    e0 = jnp.exp(z0 - jnp.max(z0, axis=1, keepdims=True))
    e1 = jnp.exp(z1 - jnp.max(z1, axis=1, keepdims=True))
    y0 = e0 / jnp.sum(e0, axis=1, keepdims=True)
    y1 = e1 / jnp.sum(e1, axis=1, keepdims=True)
    pad = jnp.zeros((z.shape[0], D - CODE - 1), _f32)
    y_ref[...] = jnp.concatenate([y0, y1, invdeg, pad], axis=1)


def _code(acc1, r, bl1, u):
    blk = 1000
    return pl.pallas_call(
        _code_body,
        grid=(N // blk,),
        in_specs=[
            pl.BlockSpec((NC, blk, D), lambda i: (0, i, 0)),
            pl.BlockSpec((blk, CODE), lambda i: (i, 0)),
            pl.BlockSpec((1, CODE), lambda i: (0, 0)),
            pl.BlockSpec((blk, CODE), lambda i: (i, 0)),
        ],
        out_specs=pl.BlockSpec((blk, D), lambda i: (i, 0)),
        out_shape=jax.ShapeDtypeStruct((N, D), _f32),
    )(acc1, r, bl1, u)


def _dec_body(acc_ref, y_ref, wl_ref, wr_ref, bl_ref, out_ref):
    acc = acc_ref[0] + acc_ref[1]
    ypad = y_ref[...]
    invdeg = ypad[:, CODE:CODE + 1]
    mean2 = acc[:, :CODE] * invdeg
    out_ref[...] = (jnp.dot(mean2, wl_ref[...], preferred_element_type=_f32)
                    + jnp.dot(ypad[:, :CODE], wr_ref[...],
                              preferred_element_type=_f32)
                    + bl_ref[...])


def _dec(acc2, y_pad, wl2_t, wr2_t, bl2):
    blk = 1000
    return pl.pallas_call(
        _dec_body,
        grid=(N // blk,),
        in_specs=[
            pl.BlockSpec((NC, blk, D), lambda i: (0, i, 0)),
            pl.BlockSpec((blk, D), lambda i: (i, 0)),
            pl.BlockSpec((CODE, IN_DIM), lambda i: (0, 0)),
            pl.BlockSpec((CODE, IN_DIM), lambda i: (0, 0)),
            pl.BlockSpec((1, IN_DIM), lambda i: (0, 0)),
        ],
        out_specs=pl.BlockSpec((blk, IN_DIM), lambda i: (i, 0)),
        out_shape=jax.ShapeDtypeStruct((N, IN_DIM), _f32),
    )(acc2, y_pad, wl2_t, wr2_t, bl2)


def kernel(x, edge_index, Wl1, bl1, Wr1, Wl2, bl2, Wr2):
    src = edge_index[0].astype(jnp.int32).reshape(NC * NS, NCH, CHUNK)
    dst = edge_index[1].astype(jnp.int32).reshape(NC * NS, NCH, CHUNK)

    wl_pad_t = jnp.zeros((IN_DIM, D), _f32).at[:, :CODE].set(Wl1.T)
    p_pad, r = _proj(x, wl_pad_t, Wr1.T)

    zeros = jnp.zeros((N, D), _f32)
    acc1 = _segsum(p_pad, src, dst, zeros)

    u = jax.random.uniform(jax.random.key(123), (N, 2, K),
                           dtype=_f32).reshape(N, CODE)
    y_pad = _code(acc1, r, bl1.reshape(1, CODE), u)

    acc2 = _segsum(y_pad, src, dst, zeros)

    return _dec(acc2, y_pad, Wl2.T, Wr2.T, bl2.reshape(1, IN_DIM))


# speedup vs baseline: 18.5472x; 1.3926x over previous
import functools

import jax
import jax.numpy as jnp
from jax import lax
from jax.experimental import pallas as pl
from jax.experimental.pallas import tpu as pltpu
from jax.experimental.pallas import tpu_sc as plsc

N = 10000
E = 320000
IN_DIM = 128
CODE = 20
K = 10
D = 32
EPS = 1e-20

NC = 2
NS = 16
E_PER_SC = E // NC
E_PER_TILE = E_PER_SC // NS
CHUNK = 80
NCH = E_PER_TILE // CHUNK
NBUF = 4
ROWS_A = 624
ROWS_LAST = N - 15 * ROWS_A

_f32 = jnp.float32


def _proj_body(x_ref, wl_ref, wr_ref, p_ref, r_ref):
    xb = x_ref[...]
    p = jnp.dot(xb, wl_ref[...], preferred_element_type=_f32)
    col = lax.broadcasted_iota(jnp.int32, p.shape, 1)
    p_ref[...] = p + jnp.where(col == CODE, _f32(1.0), _f32(0.0))
    r_ref[...] = jnp.dot(xb, wr_ref[...], preferred_element_type=_f32)


def _proj(x, wl_pad_t, wr_t):
    blk = 1000
    return pl.pallas_call(
        _proj_body,
        grid=(N // blk,),
        in_specs=[
            pl.BlockSpec((blk, IN_DIM), lambda i: (i, 0)),
            pl.BlockSpec((IN_DIM, D), lambda i: (0, 0)),
            pl.BlockSpec((IN_DIM, CODE), lambda i: (0, 0)),
        ],
        out_specs=[
            pl.BlockSpec((blk, D), lambda i: (i, 0)),
            pl.BlockSpec((blk, CODE), lambda i: (i, 0)),
        ],
        out_shape=[
            jax.ShapeDtypeStruct((N, D), _f32),
            jax.ShapeDtypeStruct((N, CODE), _f32),
        ],
    )(x, wl_pad_t, wr_t)


def _segsum_body(vals_hbm, src_hbm, dst_hbm, zeros_hbm, out_hbm,
                 srcs_v, dsts_v, rows_v, acc_sh, gsem, ssem):
    cid = lax.axis_index("c")
    sid = lax.axis_index("s")
    tid = cid * NS + sid
    row0 = pl.multiple_of(sid * ROWS_A, 8)

    @pl.when(sid < NS - 1)
    def _():
        pltpu.sync_copy(zeros_hbm.at[pl.ds(row0, ROWS_A)],
                        acc_sh.at[pl.ds(row0, ROWS_A)])

    @pl.when(sid == NS - 1)
    def _():
        pltpu.sync_copy(zeros_hbm.at[pl.ds((NS - 1) * ROWS_A, ROWS_LAST)],
                        acc_sh.at[pl.ds((NS - 1) * ROWS_A, ROWS_LAST)])

    pltpu.sync_copy(src_hbm.at[tid], srcs_v)
    pltpu.sync_copy(dst_hbm.at[tid], dsts_v)
    plsc.subcore_barrier()

    pltpu.async_copy(vals_hbm.at[srcs_v.at[0]], rows_v.at[0], gsem.at[0])
    pltpu.async_copy(vals_hbm.at[srcs_v.at[1]], rows_v.at[1], gsem.at[1])

    def body(j, carry):
        b = lax.rem(j, NBUF)
        pltpu.make_async_copy(vals_hbm.at[srcs_v.at[j]], rows_v.at[b],
                              gsem.at[b]).wait()
        pltpu.async_copy(rows_v.at[b], acc_sh.at[dsts_v.at[j]], ssem.at[b],
                         add=True)
        t = j + 2
        tb = lax.rem(t, NBUF)

        @pl.when(jnp.logical_and(t >= NBUF, t < NCH))
        def _():
            pltpu.make_async_copy(rows_v.at[tb],
                                  acc_sh.at[dsts_v.at[t - NBUF]],
                                  ssem.at[tb]).wait()

        @pl.when(t < NCH)
        def _():
            pltpu.async_copy(vals_hbm.at[srcs_v.at[t]], rows_v.at[tb],
                             gsem.at[tb])

        return carry

    lax.fori_loop(0, NCH, body, 0)
    for t in range(NCH - NBUF, NCH):
        pltpu.make_async_copy(rows_v.at[t % NBUF], acc_sh.at[dsts_v.at[t]],
                              ssem.at[t % NBUF]).wait()
    plsc.subcore_barrier()

    @pl.when(sid < NS - 1)
    def _():
        pltpu.sync_copy(acc_sh.at[pl.ds(row0, ROWS_A)],
                        out_hbm.at[cid, pl.ds(row0, ROWS_A)])

    @pl.when(sid == NS - 1)
    def _():
        pltpu.sync_copy(acc_sh.at[pl.ds((NS - 1) * ROWS_A, ROWS_LAST)],
                        out_hbm.at[cid, pl.ds((NS - 1) * ROWS_A, ROWS_LAST)])


@functools.lru_cache(maxsize=1)
def _get_segsum():
    mesh = plsc.VectorSubcoreMesh(
        core_axis_name="c", subcore_axis_name="s",
        num_cores=NC, num_subcores=NS)
    return pl.kernel(
        _segsum_body,
        out_type=jax.ShapeDtypeStruct((NC, N, D), _f32),
        mesh=mesh,
        compiler_params=pltpu.CompilerParams(use_tc_tiling_on_sc=False),
        scratch_types=[
            pltpu.VMEM((NCH, CHUNK), jnp.int32),
            pltpu.VMEM((NCH, CHUNK), jnp.int32),
            pltpu.VMEM((NBUF, CHUNK, D), _f32),
            pltpu.VMEM_SHARED((N, D), _f32),
            pltpu.SemaphoreType.DMA((NBUF,)),
            pltpu.SemaphoreType.DMA((NBUF,)),
        ],
    )


def _segsum(vals, src3, dst3, zeros):
    return _get_segsum()(vals, src3, dst3, zeros)


def _code_body(acc_ref, r_ref, bl_ref, u_ref, y_ref):
    acc = acc_ref[0] + acc_ref[1]
    deg = acc[:, CODE:CODE + 1]
    invdeg = _f32(1.0) / jnp.maximum(deg, _f32(1.0))
    h = acc[:, :CODE] * invdeg + bl_ref[...] + r_ref[...]
    g = -jnp.log(-jnp.log(u_ref[...] + _f32(EPS)))
    z = h + g
    z0 = z[:, :K]
    z1 = z[:, K:]
    e0 = jnp.exp(z0 - jnp.max(z0, axis=1, keepdims=True))
    e1 = jnp.exp(z1 - jnp.max(z1, axis=1, keepdims=True))
    y0 = e0 / jnp.sum(e0, axis=1, keepdims=True)
    y1 = e1 / jnp.sum(e1, axis=1, keepdims=True)
    pad = jnp.zeros((z.shape[0], D - CODE - 1), _f32)
    y_ref[...] = jnp.concatenate([y0, y1, invdeg, pad], axis=1)


def _code(acc1, r, bl1, u):
    blk = 1000
    return pl.pallas_call(
        _code_body,
        grid=(N // blk,),
        in_specs=[
            pl.BlockSpec((NC, blk, D), lambda i: (0, i, 0)),
            pl.BlockSpec((blk, CODE), lambda i: (i, 0)),
            pl.BlockSpec((1, CODE), lambda i: (0, 0)),
            pl.BlockSpec((blk, CODE), lambda i: (i, 0)),
        ],
        out_specs=pl.BlockSpec((blk, D), lambda i: (i, 0)),
        out_shape=jax.ShapeDtypeStruct((N, D), _f32),
    )(acc1, r, bl1, u)


def _dec_body(acc_ref, y_ref, wl_ref, wr_ref, bl_ref, out_ref):
    acc = acc_ref[0] + acc_ref[1]
    ypad = y_ref[...]
    invdeg = ypad[:, CODE:CODE + 1]
    mean2 = acc[:, :CODE] * invdeg
    out_ref[...] = (jnp.dot(mean2, wl_ref[...], preferred_element_type=_f32)
                    + jnp.dot(ypad[:, :CODE], wr_ref[...],
                              preferred_element_type=_f32)
                    + bl_ref[...])


def _dec(acc2, y_pad, wl2_t, wr2_t, bl2):
    blk = 1000
    return pl.pallas_call(
        _dec_body,
        grid=(N // blk,),
        in_specs=[
            pl.BlockSpec((NC, blk, D), lambda i: (0, i, 0)),
            pl.BlockSpec((blk, D), lambda i: (i, 0)),
            pl.BlockSpec((CODE, IN_DIM), lambda i: (0, 0)),
            pl.BlockSpec((CODE, IN_DIM), lambda i: (0, 0)),
            pl.BlockSpec((1, IN_DIM), lambda i: (0, 0)),
        ],
        out_specs=pl.BlockSpec((blk, IN_DIM), lambda i: (i, 0)),
        out_shape=jax.ShapeDtypeStruct((N, IN_DIM), _f32),
    )(acc2, y_pad, wl2_t, wr2_t, bl2)


def kernel(x, edge_index, Wl1, bl1, Wr1, Wl2, bl2, Wr2):
    src = edge_index[0].astype(jnp.int32).reshape(NC * NS, NCH, CHUNK)
    dst = edge_index[1].astype(jnp.int32).reshape(NC * NS, NCH, CHUNK)

    wl_pad_t = jnp.zeros((IN_DIM, D), _f32).at[:, :CODE].set(Wl1.T)
    p_pad, r = _proj(x, wl_pad_t, Wr1.T)

    zeros = jnp.zeros((N, D), _f32)
    acc1 = _segsum(p_pad, src, dst, zeros)

    u = jax.random.uniform(jax.random.key(123), (N, 2, K),
                           dtype=_f32).reshape(N, CODE)
    y_pad = _code(acc1, r, bl1.reshape(1, CODE), u)

    acc2 = _segsum(y_pad, src, dst, zeros)

    return _dec(acc2, y_pad, Wl2.T, Wr2.T, bl2.reshape(1, IN_DIM))


# speedup vs baseline: 22.3874x; 1.2070x over previous
import functools

import jax
import jax.numpy as jnp
from jax import lax
from jax.experimental import pallas as pl
from jax.experimental.pallas import tpu as pltpu
from jax.experimental.pallas import tpu_sc as plsc

N = 10000
E = 320000
IN_DIM = 128
CODE = 20
K = 10
D = 32
EPS = 1e-20

NC = 2
NS = 16
E_PER_SC = E // NC
E_PER_TILE = E_PER_SC // NS
CHUNK = 80
NCH = E_PER_TILE // CHUNK
NBUF = 8
LOOKAHEAD = 4
ROWS_A = 624
ROWS_LAST = N - 15 * ROWS_A

_f32 = jnp.float32


def _proj_body(x_ref, wl_ref, wr_ref, p_ref, r_ref):
    xb = x_ref[...]
    p = jnp.dot(xb, wl_ref[...], preferred_element_type=_f32)
    col = lax.broadcasted_iota(jnp.int32, p.shape, 1)
    p_ref[...] = p + jnp.where(col == CODE, _f32(1.0), _f32(0.0))
    r_ref[...] = jnp.dot(xb, wr_ref[...], preferred_element_type=_f32)


def _proj(x, wl_pad_t, wr_t):
    blk = 1000
    return pl.pallas_call(
        _proj_body,
        grid=(N // blk,),
        in_specs=[
            pl.BlockSpec((blk, IN_DIM), lambda i: (i, 0)),
            pl.BlockSpec((IN_DIM, D), lambda i: (0, 0)),
            pl.BlockSpec((IN_DIM, CODE), lambda i: (0, 0)),
        ],
        out_specs=[
            pl.BlockSpec((blk, D), lambda i: (i, 0)),
            pl.BlockSpec((blk, CODE), lambda i: (i, 0)),
        ],
        out_shape=[
            jax.ShapeDtypeStruct((N, D), _f32),
            jax.ShapeDtypeStruct((N, CODE), _f32),
        ],
    )(x, wl_pad_t, wr_t)


def _segsum_body(vals_hbm, src_hbm, dst_hbm, zeros_hbm, out_hbm,
                 srcs_v, dsts_v, rows_v, acc_sh, gsem, ssem):
    cid = lax.axis_index("c")
    sid = lax.axis_index("s")
    tid = cid * NS + sid
    row0 = pl.multiple_of(sid * ROWS_A, 8)

    @pl.when(sid < NS - 1)
    def _():
        pltpu.sync_copy(zeros_hbm.at[pl.ds(row0, ROWS_A)],
                        acc_sh.at[pl.ds(row0, ROWS_A)])

    @pl.when(sid == NS - 1)
    def _():
        pltpu.sync_copy(zeros_hbm.at[pl.ds((NS - 1) * ROWS_A, ROWS_LAST)],
                        acc_sh.at[pl.ds((NS - 1) * ROWS_A, ROWS_LAST)])

    pltpu.sync_copy(src_hbm.at[tid], srcs_v)
    pltpu.sync_copy(dst_hbm.at[tid], dsts_v)
    plsc.subcore_barrier()

    for k in range(LOOKAHEAD):
        pltpu.async_copy(vals_hbm.at[srcs_v.at[k]], rows_v.at[k], gsem.at[k])

    def body(j, carry):
        b = lax.rem(j, NBUF)
        pltpu.make_async_copy(vals_hbm.at[srcs_v.at[j]], rows_v.at[b],
                              gsem.at[b]).wait()
        pltpu.async_copy(rows_v.at[b], acc_sh.at[dsts_v.at[j]], ssem.at[b],
                         add=True)
        t = j + LOOKAHEAD
        tb = lax.rem(t, NBUF)

        @pl.when(jnp.logical_and(t >= NBUF, t < NCH))
        def _():
            pltpu.make_async_copy(rows_v.at[tb],
                                  acc_sh.at[dsts_v.at[t - NBUF]],
                                  ssem.at[tb]).wait()

        @pl.when(t < NCH)
        def _():
            pltpu.async_copy(vals_hbm.at[srcs_v.at[t]], rows_v.at[tb],
                             gsem.at[tb])

        return carry

    lax.fori_loop(0, NCH, body, 0)
    for t in range(NCH - NBUF, NCH):
        pltpu.make_async_copy(rows_v.at[t % NBUF], acc_sh.at[dsts_v.at[t]],
                              ssem.at[t % NBUF]).wait()
    plsc.subcore_barrier()

    @pl.when(sid < NS - 1)
    def _():
        pltpu.sync_copy(acc_sh.at[pl.ds(row0, ROWS_A)],
                        out_hbm.at[cid, pl.ds(row0, ROWS_A)])

    @pl.when(sid == NS - 1)
    def _():
        pltpu.sync_copy(acc_sh.at[pl.ds((NS - 1) * ROWS_A, ROWS_LAST)],
                        out_hbm.at[cid, pl.ds((NS - 1) * ROWS_A, ROWS_LAST)])


@functools.lru_cache(maxsize=1)
def _get_segsum():
    mesh = plsc.VectorSubcoreMesh(
        core_axis_name="c", subcore_axis_name="s",
        num_cores=NC, num_subcores=NS)
    return pl.kernel(
        _segsum_body,
        out_type=jax.ShapeDtypeStruct((NC, N, D), _f32),
        mesh=mesh,
        compiler_params=pltpu.CompilerParams(use_tc_tiling_on_sc=False),
        scratch_types=[
            pltpu.VMEM((NCH, CHUNK), jnp.int32),
            pltpu.VMEM((NCH, CHUNK), jnp.int32),
            pltpu.VMEM((NBUF, CHUNK, D), _f32),
            pltpu.VMEM_SHARED((N, D), _f32),
            pltpu.SemaphoreType.DMA((NBUF,)),
            pltpu.SemaphoreType.DMA((NBUF,)),
        ],
    )


def _segsum(vals, src3, dst3, zeros):
    return _get_segsum()(vals, src3, dst3, zeros)


def _code_body(acc_ref, r_ref, bl_ref, u_ref, y_ref):
    acc = acc_ref[0] + acc_ref[1]
    deg = acc[:, CODE:CODE + 1]
    invdeg = _f32(1.0) / jnp.maximum(deg, _f32(1.0))
    h = acc[:, :CODE] * invdeg + bl_ref[...] + r_ref[...]
    g = -jnp.log(-jnp.log(u_ref[...] + _f32(EPS)))
    z = h + g
    z0 = z[:, :K]
    z1 = z[:, K:]
    e0 = jnp.exp(z0 - jnp.max(z0, axis=1, keepdims=True))
    e1 = jnp.exp(z1 - jnp.max(z1, axis=1, keepdims=True))
    y0 = e0 / jnp.sum(e0, axis=1, keepdims=True)
    y1 = e1 / jnp.sum(e1, axis=1, keepdims=True)
    pad = jnp.zeros((z.shape[0], D - CODE - 1), _f32)
    y_ref[...] = jnp.concatenate([y0, y1, invdeg, pad], axis=1)


def _code(acc1, r, bl1, u):
    blk = 1000
    return pl.pallas_call(
        _code_body,
        grid=(N // blk,),
        in_specs=[
            pl.BlockSpec((NC, blk, D), lambda i: (0, i, 0)),
            pl.BlockSpec((blk, CODE), lambda i: (i, 0)),
            pl.BlockSpec((1, CODE), lambda i: (0, 0)),
            pl.BlockSpec((blk, CODE), lambda i: (i, 0)),
        ],
        out_specs=pl.BlockSpec((blk, D), lambda i: (i, 0)),
        out_shape=jax.ShapeDtypeStruct((N, D), _f32),
    )(acc1, r, bl1, u)


def _dec_body(acc_ref, y_ref, wl_ref, wr_ref, bl_ref, out_ref):
    acc = acc_ref[0] + acc_ref[1]
    ypad = y_ref[...]
    invdeg = ypad[:, CODE:CODE + 1]
    mean2 = acc[:, :CODE] * invdeg
    out_ref[...] = (jnp.dot(mean2, wl_ref[...], preferred_element_type=_f32)
                    + jnp.dot(ypad[:, :CODE], wr_ref[...],
                              preferred_element_type=_f32)
                    + bl_ref[...])


def _dec(acc2, y_pad, wl2_t, wr2_t, bl2):
    blk = 1000
    return pl.pallas_call(
        _dec_body,
        grid=(N // blk,),
        in_specs=[
            pl.BlockSpec((NC, blk, D), lambda i: (0, i, 0)),
            pl.BlockSpec((blk, D), lambda i: (i, 0)),
            pl.BlockSpec((CODE, IN_DIM), lambda i: (0, 0)),
            pl.BlockSpec((CODE, IN_DIM), lambda i: (0, 0)),
            pl.BlockSpec((1, IN_DIM), lambda i: (0, 0)),
        ],
        out_specs=pl.BlockSpec((blk, IN_DIM), lambda i: (i, 0)),
        out_shape=jax.ShapeDtypeStruct((N, IN_DIM), _f32),
    )(acc2, y_pad, wl2_t, wr2_t, bl2)


def kernel(x, edge_index, Wl1, bl1, Wr1, Wl2, bl2, Wr2):
    src = edge_index[0].astype(jnp.int32).reshape(NC * NS, NCH, CHUNK)
    dst = edge_index[1].astype(jnp.int32).reshape(NC * NS, NCH, CHUNK)

    wl_pad_t = jnp.zeros((IN_DIM, D), _f32).at[:, :CODE].set(Wl1.T)
    p_pad, r = _proj(x, wl_pad_t, Wr1.T)

    zeros = jnp.zeros((N, D), _f32)
    acc1 = _segsum(p_pad, src, dst, zeros)

    u = jax.random.uniform(jax.random.key(123), (N, 2, K),
                           dtype=_f32).reshape(N, CODE)
    y_pad = _code(acc1, r, bl1.reshape(1, CODE), u)

    acc2 = _segsum(y_pad, src, dst, zeros)

    return _dec(acc2, y_pad, Wl2.T, Wr2.T, bl2.reshape(1, IN_DIM))


# speedup vs baseline: 23.6966x; 1.0585x over previous
import functools

import jax
import jax.numpy as jnp
from jax import lax
from jax.experimental import pallas as pl
from jax.experimental.pallas import tpu as pltpu
from jax.experimental.pallas import tpu_sc as plsc

N = 10000
E = 320000
IN_DIM = 128
CODE = 20
K = 10
D = 32
EPS = 1e-20

NC = 2
NS = 16
E_PER_SC = E // NC
E_PER_TILE = E_PER_SC // NS
CHUNK = 80
NCH = E_PER_TILE // CHUNK
NBUF = 12
LOOKAHEAD = 6
ROWS_A = 624
ROWS_LAST = N - 15 * ROWS_A

_f32 = jnp.float32


def _proj_body(x_ref, wl_ref, wr_ref, p_ref, r_ref):
    xb = x_ref[...]
    p = jnp.dot(xb, wl_ref[...], preferred_element_type=_f32)
    col = lax.broadcasted_iota(jnp.int32, p.shape, 1)
    p_ref[...] = p + jnp.where(col == CODE, _f32(1.0), _f32(0.0))
    r_ref[...] = jnp.dot(xb, wr_ref[...], preferred_element_type=_f32)


def _proj(x, wl_pad_t, wr_t):
    blk = 1000
    return pl.pallas_call(
        _proj_body,
        grid=(N // blk,),
        in_specs=[
            pl.BlockSpec((blk, IN_DIM), lambda i: (i, 0)),
            pl.BlockSpec((IN_DIM, D), lambda i: (0, 0)),
            pl.BlockSpec((IN_DIM, CODE), lambda i: (0, 0)),
        ],
        out_specs=[
            pl.BlockSpec((blk, D), lambda i: (i, 0)),
            pl.BlockSpec((blk, CODE), lambda i: (i, 0)),
        ],
        out_shape=[
            jax.ShapeDtypeStruct((N, D), _f32),
            jax.ShapeDtypeStruct((N, CODE), _f32),
        ],
    )(x, wl_pad_t, wr_t)


def _segsum_body(vals_hbm, src_hbm, dst_hbm, zeros_hbm, out_hbm,
                 srcs_v, dsts_v, rows_v, acc_sh, gsem, ssem):
    cid = lax.axis_index("c")
    sid = lax.axis_index("s")
    tid = cid * NS + sid
    row0 = pl.multiple_of(sid * ROWS_A, 8)

    @pl.when(sid < NS - 1)
    def _():
        pltpu.sync_copy(zeros_hbm.at[pl.ds(row0, ROWS_A)],
                        acc_sh.at[pl.ds(row0, ROWS_A)])

    @pl.when(sid == NS - 1)
    def _():
        pltpu.sync_copy(zeros_hbm.at[pl.ds((NS - 1) * ROWS_A, ROWS_LAST)],
                        acc_sh.at[pl.ds((NS - 1) * ROWS_A, ROWS_LAST)])

    pltpu.sync_copy(src_hbm.at[tid], srcs_v)
    pltpu.sync_copy(dst_hbm.at[tid], dsts_v)
    plsc.subcore_barrier()

    for k in range(LOOKAHEAD):
        pltpu.async_copy(vals_hbm.at[srcs_v.at[k]], rows_v.at[k], gsem.at[k])

    def body(j, carry):
        b = lax.rem(j, NBUF)
        pltpu.make_async_copy(vals_hbm.at[srcs_v.at[j]], rows_v.at[b],
                              gsem.at[b]).wait()
        pltpu.async_copy(rows_v.at[b], acc_sh.at[dsts_v.at[j]], ssem.at[b],
                         add=True)
        t = j + LOOKAHEAD
        tb = lax.rem(t, NBUF)

        @pl.when(jnp.logical_and(t >= NBUF, t < NCH))
        def _():
            pltpu.make_async_copy(rows_v.at[tb],
                                  acc_sh.at[dsts_v.at[t - NBUF]],
                                  ssem.at[tb]).wait()

        @pl.when(t < NCH)
        def _():
            pltpu.async_copy(vals_hbm.at[srcs_v.at[t]], rows_v.at[tb],
                             gsem.at[tb])

        return carry

    lax.fori_loop(0, NCH, body, 0)
    for t in range(NCH - NBUF, NCH):
        pltpu.make_async_copy(rows_v.at[t % NBUF], acc_sh.at[dsts_v.at[t]],
                              ssem.at[t % NBUF]).wait()
    plsc.subcore_barrier()

    @pl.when(sid < NS - 1)
    def _():
        pltpu.sync_copy(acc_sh.at[pl.ds(row0, ROWS_A)],
                        out_hbm.at[cid, pl.ds(row0, ROWS_A)])

    @pl.when(sid == NS - 1)
    def _():
        pltpu.sync_copy(acc_sh.at[pl.ds((NS - 1) * ROWS_A, ROWS_LAST)],
                        out_hbm.at[cid, pl.ds((NS - 1) * ROWS_A, ROWS_LAST)])


@functools.lru_cache(maxsize=1)
def _get_segsum():
    mesh = plsc.VectorSubcoreMesh(
        core_axis_name="c", subcore_axis_name="s",
        num_cores=NC, num_subcores=NS)
    return pl.kernel(
        _segsum_body,
        out_type=jax.ShapeDtypeStruct((NC, N, D), _f32),
        mesh=mesh,
        compiler_params=pltpu.CompilerParams(use_tc_tiling_on_sc=False),
        scratch_types=[
            pltpu.VMEM((NCH, CHUNK), jnp.int32),
            pltpu.VMEM((NCH, CHUNK), jnp.int32),
            pltpu.VMEM((NBUF, CHUNK, D), _f32),
            pltpu.VMEM_SHARED((N, D), _f32),
            pltpu.SemaphoreType.DMA((NBUF,)),
            pltpu.SemaphoreType.DMA((NBUF,)),
        ],
    )


def _segsum(vals, src3, dst3, zeros):
    return _get_segsum()(vals, src3, dst3, zeros)


def _code_body(acc_ref, r_ref, bl_ref, u_ref, y_ref):
    acc = acc_ref[0] + acc_ref[1]
    deg = acc[:, CODE:CODE + 1]
    invdeg = _f32(1.0) / jnp.maximum(deg, _f32(1.0))
    h = acc[:, :CODE] * invdeg + bl_ref[...] + r_ref[...]
    g = -jnp.log(-jnp.log(u_ref[...] + _f32(EPS)))
    z = h + g
    z0 = z[:, :K]
    z1 = z[:, K:]
    e0 = jnp.exp(z0 - jnp.max(z0, axis=1, keepdims=True))
    e1 = jnp.exp(z1 - jnp.max(z1, axis=1, keepdims=True))
    y0 = e0 / jnp.sum(e0, axis=1, keepdims=True)
    y1 = e1 / jnp.sum(e1, axis=1, keepdims=True)
    pad = jnp.zeros((z.shape[0], D - CODE - 1), _f32)
    y_ref[...] = jnp.concatenate([y0, y1, invdeg, pad], axis=1)


def _code(acc1, r, bl1, u):
    blk = 1000
    return pl.pallas_call(
        _code_body,
        grid=(N // blk,),
        in_specs=[
            pl.BlockSpec((NC, blk, D), lambda i: (0, i, 0)),
            pl.BlockSpec((blk, CODE), lambda i: (i, 0)),
            pl.BlockSpec((1, CODE), lambda i: (0, 0)),
            pl.BlockSpec((blk, CODE), lambda i: (i, 0)),
        ],
        out_specs=pl.BlockSpec((blk, D), lambda i: (i, 0)),
        out_shape=jax.ShapeDtypeStruct((N, D), _f32),
    )(acc1, r, bl1, u)


def _dec_body(acc_ref, y_ref, wl_ref, wr_ref, bl_ref, out_ref):
    acc = acc_ref[0] + acc_ref[1]
    ypad = y_ref[...]
    invdeg = ypad[:, CODE:CODE + 1]
    mean2 = acc[:, :CODE] * invdeg
    out_ref[...] = (jnp.dot(mean2, wl_ref[...], preferred_element_type=_f32)
                    + jnp.dot(ypad[:, :CODE], wr_ref[...],
                              preferred_element_type=_f32)
                    + bl_ref[...])


def _dec(acc2, y_pad, wl2_t, wr2_t, bl2):
    blk = 1000
    return pl.pallas_call(
        _dec_body,
        grid=(N // blk,),
        in_specs=[
            pl.BlockSpec((NC, blk, D), lambda i: (0, i, 0)),
            pl.BlockSpec((blk, D), lambda i: (i, 0)),
            pl.BlockSpec((CODE, IN_DIM), lambda i: (0, 0)),
            pl.BlockSpec((CODE, IN_DIM), lambda i: (0, 0)),
            pl.BlockSpec((1, IN_DIM), lambda i: (0, 0)),
        ],
        out_specs=pl.BlockSpec((blk, IN_DIM), lambda i: (i, 0)),
        out_shape=jax.ShapeDtypeStruct((N, IN_DIM), _f32),
    )(acc2, y_pad, wl2_t, wr2_t, bl2)


def kernel(x, edge_index, Wl1, bl1, Wr1, Wl2, bl2, Wr2):
    src = edge_index[0].astype(jnp.int32).reshape(NC * NS, NCH, CHUNK)
    dst = edge_index[1].astype(jnp.int32).reshape(NC * NS, NCH, CHUNK)

    wl_pad_t = jnp.zeros((IN_DIM, D), _f32).at[:, :CODE].set(Wl1.T)
    p_pad, r = _proj(x, wl_pad_t, Wr1.T)

    zeros = jnp.zeros((N, D), _f32)
    acc1 = _segsum(p_pad, src, dst, zeros)

    u = jax.random.uniform(jax.random.key(123), (N, 2, K),
                           dtype=_f32).reshape(N, CODE)
    y_pad = _code(acc1, r, bl1.reshape(1, CODE), u)

    acc2 = _segsum(y_pad, src, dst, zeros)

    return _dec(acc2, y_pad, Wl2.T, Wr2.T, bl2.reshape(1, IN_DIM))


# speedup vs baseline: 25.3778x; 1.0709x over previous
import functools

import jax
import jax.numpy as jnp
from jax import lax
from jax.experimental import pallas as pl
from jax.experimental.pallas import tpu as pltpu
from jax.experimental.pallas import tpu_sc as plsc

N = 10000
E = 320000
IN_DIM = 128
CODE = 20
K = 10
D = 32
EPS = 1e-20

NC = 2
NS = 16
E_PER_SC = E // NC
E_PER_TILE = E_PER_SC // NS
CHUNK = 80
NCH = E_PER_TILE // CHUNK
NBUF = 12
LOOKAHEAD = 8
ROWS_A = 624
ROWS_LAST = N - 15 * ROWS_A

_f32 = jnp.float32


_DN = (((1,), (1,)), ((), ()))


def _proj_body(x_ref, wl_ref, wr_ref, p_ref, r_ref):
    xb = x_ref[...]
    p = lax.dot_general(xb, wl_ref[...], _DN, preferred_element_type=_f32)
    blk = p.shape[0]
    p_ref[...] = jnp.concatenate(
        [p, jnp.ones((blk, 1), _f32), jnp.zeros((blk, D - CODE - 1), _f32)],
        axis=1)
    r_ref[...] = lax.dot_general(xb, wr_ref[...], _DN,
                                 preferred_element_type=_f32)


def _proj(x, wl, wr):
    blk = 1000
    return pl.pallas_call(
        _proj_body,
        grid=(N // blk,),
        in_specs=[
            pl.BlockSpec((blk, IN_DIM), lambda i: (i, 0)),
            pl.BlockSpec((CODE, IN_DIM), lambda i: (0, 0)),
            pl.BlockSpec((CODE, IN_DIM), lambda i: (0, 0)),
        ],
        out_specs=[
            pl.BlockSpec((blk, D), lambda i: (i, 0)),
            pl.BlockSpec((blk, CODE), lambda i: (i, 0)),
        ],
        out_shape=[
            jax.ShapeDtypeStruct((N, D), _f32),
            jax.ShapeDtypeStruct((N, CODE), _f32),
        ],
    )(x, wl, wr)


def _segsum_body(vals_hbm, src_hbm, dst_hbm, zeros_hbm, out_hbm,
                 srcs_v, dsts_v, rows_v, acc_sh, gsem, ssem, isem):
    cid = lax.axis_index("c")
    sid = lax.axis_index("s")
    tid = cid * NS + sid
    row0 = pl.multiple_of(sid * ROWS_A, 8)

    isrc = pltpu.async_copy(src_hbm.at[tid], srcs_v, isem.at[0])
    idst = pltpu.async_copy(dst_hbm.at[tid], dsts_v, isem.at[1])
    isrc.wait()

    for k in range(LOOKAHEAD):
        pltpu.async_copy(vals_hbm.at[srcs_v.at[k]], rows_v.at[k], gsem.at[k])

    @pl.when(sid < NS - 1)
    def _():
        pltpu.sync_copy(zeros_hbm.at[pl.ds(row0, ROWS_A)],
                        acc_sh.at[pl.ds(row0, ROWS_A)])

    @pl.when(sid == NS - 1)
    def _():
        pltpu.sync_copy(zeros_hbm.at[pl.ds((NS - 1) * ROWS_A, ROWS_LAST)],
                        acc_sh.at[pl.ds((NS - 1) * ROWS_A, ROWS_LAST)])

    idst.wait()
    plsc.subcore_barrier()

    def body(j, carry):
        b = lax.rem(j, NBUF)
        pltpu.make_async_copy(vals_hbm.at[srcs_v.at[j]], rows_v.at[b],
                              gsem.at[b]).wait()
        pltpu.async_copy(rows_v.at[b], acc_sh.at[dsts_v.at[j]],
                         ssem.at[b], add=True)
        t = j + LOOKAHEAD
        tb = lax.rem(t, NBUF)

        @pl.when(jnp.logical_and(t >= NBUF, t < NCH))
        def _():
            pltpu.make_async_copy(rows_v.at[tb],
                                  acc_sh.at[dsts_v.at[t - NBUF]],
                                  ssem.at[tb]).wait()

        @pl.when(t < NCH)
        def _():
            pltpu.async_copy(vals_hbm.at[srcs_v.at[t]], rows_v.at[tb],
                             gsem.at[tb])

        return carry

    lax.fori_loop(0, NCH, body, 0)
    for t in range(NCH - NBUF, NCH):
        pltpu.make_async_copy(rows_v.at[t % NBUF],
                              acc_sh.at[dsts_v.at[t]],
                              ssem.at[t % NBUF]).wait()
    plsc.subcore_barrier()

    @pl.when(sid < NS - 1)
    def _():
        pltpu.sync_copy(acc_sh.at[pl.ds(row0, ROWS_A)],
                        out_hbm.at[cid, pl.ds(row0, ROWS_A)])

    @pl.when(sid == NS - 1)
    def _():
        pltpu.sync_copy(acc_sh.at[pl.ds((NS - 1) * ROWS_A, ROWS_LAST)],
                        out_hbm.at[cid, pl.ds((NS - 1) * ROWS_A, ROWS_LAST)])


@functools.lru_cache(maxsize=1)
def _get_segsum():
    mesh = plsc.VectorSubcoreMesh(
        core_axis_name="c", subcore_axis_name="s",
        num_cores=NC, num_subcores=NS)
    return pl.kernel(
        _segsum_body,
        out_type=jax.ShapeDtypeStruct((NC, N, D), _f32),
        mesh=mesh,
        compiler_params=pltpu.CompilerParams(use_tc_tiling_on_sc=False),
        scratch_types=[
            pltpu.VMEM((NCH, CHUNK), jnp.int32),
            pltpu.VMEM((NCH, CHUNK), jnp.int32),
            pltpu.VMEM((NBUF, CHUNK, D), _f32),
            pltpu.VMEM_SHARED((N, D), _f32),
            pltpu.SemaphoreType.DMA((NBUF,)),
            pltpu.SemaphoreType.DMA((NBUF,)),
            pltpu.SemaphoreType.DMA((2,)),
        ],
    )


def _segsum(vals, src3, dst3, zeros):
    return _get_segsum()(vals, src3, dst3, zeros)


def _code_body(acc_ref, r_ref, bl_ref, u_ref, y_ref):
    acc = acc_ref[0] + acc_ref[1]
    deg = acc[:, CODE:CODE + 1]
    invdeg = _f32(1.0) / jnp.maximum(deg, _f32(1.0))
    h = acc[:, :CODE] * invdeg + bl_ref[...] + r_ref[...]
    g = -jnp.log(-jnp.log(u_ref[...] + _f32(EPS)))
    z = h + g
    z0 = z[:, :K]
    z1 = z[:, K:]
    e0 = jnp.exp(z0 - jnp.max(z0, axis=1, keepdims=True))
    e1 = jnp.exp(z1 - jnp.max(z1, axis=1, keepdims=True))
    y0 = e0 / jnp.sum(e0, axis=1, keepdims=True)
    y1 = e1 / jnp.sum(e1, axis=1, keepdims=True)
    pad = jnp.zeros((z.shape[0], D - CODE - 1), _f32)
    y_ref[...] = jnp.concatenate([y0, y1, invdeg, pad], axis=1)


def _code(acc1, r, bl1, u):
    blk = 1000
    return pl.pallas_call(
        _code_body,
        grid=(N // blk,),
        in_specs=[
            pl.BlockSpec((NC, blk, D), lambda i: (0, i, 0)),
            pl.BlockSpec((blk, CODE), lambda i: (i, 0)),
            pl.BlockSpec((1, CODE), lambda i: (0, 0)),
            pl.BlockSpec((blk, CODE), lambda i: (i, 0)),
        ],
        out_specs=pl.BlockSpec((blk, D), lambda i: (i, 0)),
        out_shape=jax.ShapeDtypeStruct((N, D), _f32),
    )(acc1, r, bl1, u)


def _dec_body(acc_ref, y_ref, wl_ref, wr_ref, bl_ref, out_ref):
    acc = acc_ref[0] + acc_ref[1]
    ypad = y_ref[...]
    invdeg = ypad[:, CODE:CODE + 1]
    mean2 = acc[:, :CODE] * invdeg
    out_ref[...] = (lax.dot_general(mean2, wl_ref[...], _DN,
                                    preferred_element_type=_f32)
                    + lax.dot_general(ypad[:, :CODE], wr_ref[...], _DN,
                                      preferred_element_type=_f32)
                    + bl_ref[...])


def _dec(acc2, y_pad, wl2, wr2, bl2):
    blk = 1000
    return pl.pallas_call(
        _dec_body,
        grid=(N // blk,),
        in_specs=[
            pl.BlockSpec((NC, blk, D), lambda i: (0, i, 0)),
            pl.BlockSpec((blk, D), lambda i: (i, 0)),
            pl.BlockSpec((IN_DIM, CODE), lambda i: (0, 0)),
            pl.BlockSpec((IN_DIM, CODE), lambda i: (0, 0)),
            pl.BlockSpec((1, IN_DIM), lambda i: (0, 0)),
        ],
        out_specs=pl.BlockSpec((blk, IN_DIM), lambda i: (i, 0)),
        out_shape=jax.ShapeDtypeStruct((N, IN_DIM), _f32),
    )(acc2, y_pad, wl2, wr2, bl2)


def kernel(x, edge_index, Wl1, bl1, Wr1, Wl2, bl2, Wr2):
    src = edge_index[0].astype(jnp.int32).reshape(NC * NS, NCH, CHUNK)
    dst = edge_index[1].astype(jnp.int32).reshape(NC * NS, NCH, CHUNK)

    p_pad, r = _proj(x, Wl1, Wr1)

    zeros = jnp.zeros((N, D), _f32)
    acc1 = _segsum(p_pad, src, dst, zeros)

    u = jax.random.uniform(jax.random.key(123), (N, 2, K),
                           dtype=_f32).reshape(N, CODE)
    y_pad = _code(acc1, r, bl1.reshape(1, CODE), u)

    acc2 = _segsum(y_pad, src, dst, zeros)

    return _dec(acc2, y_pad, Wl2, Wr2, bl2.reshape(1, IN_DIM))


# speedup vs baseline: 26.2788x; 1.0355x over previous
import functools

import jax
import jax.numpy as jnp
from jax import lax
from jax.experimental import pallas as pl
from jax.experimental.pallas import tpu as pltpu
from jax.experimental.pallas import tpu_sc as plsc

N = 10000
E = 320000
IN_DIM = 128
CODE = 20
K = 10
D = 32
EPS = 1e-20

NC = 2
NS = 16
E_PER_SC = E // NC
E_PER_TILE = E_PER_SC // NS
CHUNK = 80
NCH = E_PER_TILE // CHUNK
NBUF = 12
LOOKAHEAD = 8
ROWS_A = 624
ROWS_LAST = N - 15 * ROWS_A

_f32 = jnp.float32


_DN = (((1,), (1,)), ((), ()))


def _idx_body(e_ref, o_ref):
    e = e_ref[...]
    q = E // 4
    rows = [e[r // 4:r // 4 + 1, (r % 4) * q:(r % 4 + 1) * q]
            for r in range(8)]
    o_ref[...] = jnp.concatenate(rows, axis=0)


def _idx_prep(edge_index):
    return pl.pallas_call(
        _idx_body,
        out_shape=jax.ShapeDtypeStruct((8, E // 4), jnp.int32),
    )(edge_index)


def _gum_body(u_ref, g_ref):
    g_ref[...] = -jnp.log(-jnp.log(u_ref[...] + _f32(EPS)))


def _gum(u):
    blk = 2000
    return pl.pallas_call(
        _gum_body,
        grid=(N // blk,),
        in_specs=[pl.BlockSpec((blk, CODE), lambda i: (i, 0))],
        out_specs=pl.BlockSpec((blk, CODE), lambda i: (i, 0)),
        out_shape=jax.ShapeDtypeStruct((N, CODE), _f32),
    )(u)


def _proj_body(x_ref, wl_ref, wr_ref, p_ref, r_ref):
    xb = x_ref[...]
    p = lax.dot_general(xb, wl_ref[...], _DN, preferred_element_type=_f32)
    blk = p.shape[0]
    p_ref[...] = jnp.concatenate(
        [p, jnp.ones((blk, 1), _f32), jnp.zeros((blk, D - CODE - 1), _f32)],
        axis=1)
    r_ref[...] = lax.dot_general(xb, wr_ref[...], _DN,
                                 preferred_element_type=_f32)


def _proj(x, wl, wr):
    blk = 1000
    return pl.pallas_call(
        _proj_body,
        grid=(N // blk,),
        in_specs=[
            pl.BlockSpec((blk, IN_DIM), lambda i: (i, 0)),
            pl.BlockSpec((CODE, IN_DIM), lambda i: (0, 0)),
            pl.BlockSpec((CODE, IN_DIM), lambda i: (0, 0)),
        ],
        out_specs=[
            pl.BlockSpec((blk, D), lambda i: (i, 0)),
            pl.BlockSpec((blk, CODE), lambda i: (i, 0)),
        ],
        out_shape=[
            jax.ShapeDtypeStruct((N, D), _f32),
            jax.ShapeDtypeStruct((N, CODE), _f32),
        ],
    )(x, wl, wr)


def _segsum_body(vals_hbm, idx_hbm, zeros_hbm, out_hbm,
                 srcs_v, dsts_v, rows_v, acc_sh, gsem, ssem, isem):
    cid = lax.axis_index("c")
    sid = lax.axis_index("s")
    tid = cid * NS + sid
    row0 = pl.multiple_of(sid * ROWS_A, 8)

    isrc = pltpu.async_copy(idx_hbm.at[0, tid], srcs_v, isem.at[0])
    idst = pltpu.async_copy(idx_hbm.at[1, tid], dsts_v, isem.at[1])
    isrc.wait()

    for k in range(LOOKAHEAD):
        pltpu.async_copy(vals_hbm.at[srcs_v.at[k]], rows_v.at[k], gsem.at[k])

    @pl.when(sid < NS - 1)
    def _():
        pltpu.sync_copy(zeros_hbm.at[pl.ds(row0, ROWS_A)],
                        acc_sh.at[pl.ds(row0, ROWS_A)])

    @pl.when(sid == NS - 1)
    def _():
        pltpu.sync_copy(zeros_hbm.at[pl.ds((NS - 1) * ROWS_A, ROWS_LAST)],
                        acc_sh.at[pl.ds((NS - 1) * ROWS_A, ROWS_LAST)])

    idst.wait()
    plsc.subcore_barrier()

    def body(j, carry):
        b = lax.rem(j, NBUF)
        pltpu.make_async_copy(vals_hbm.at[srcs_v.at[j]], rows_v.at[b],
                              gsem.at[b]).wait()
        pltpu.async_copy(rows_v.at[b], acc_sh.at[dsts_v.at[j]],
                         ssem.at[b], add=True)
        t = j + LOOKAHEAD
        tb = lax.rem(t, NBUF)

        @pl.when(jnp.logical_and(t >= NBUF, t < NCH))
        def _():
            pltpu.make_async_copy(rows_v.at[tb],
                                  acc_sh.at[dsts_v.at[t - NBUF]],
                                  ssem.at[tb]).wait()

        @pl.when(t < NCH)
        def _():
            pltpu.async_copy(vals_hbm.at[srcs_v.at[t]], rows_v.at[tb],
                             gsem.at[tb])

        return carry

    lax.fori_loop(0, NCH, body, 0)
    for t in range(NCH - NBUF, NCH):
        pltpu.make_async_copy(rows_v.at[t % NBUF],
                              acc_sh.at[dsts_v.at[t]],
                              ssem.at[t % NBUF]).wait()
    plsc.subcore_barrier()

    @pl.when(sid < NS - 1)
    def _():
        pltpu.sync_copy(acc_sh.at[pl.ds(row0, ROWS_A)],
                        out_hbm.at[cid, pl.ds(row0, ROWS_A)])

    @pl.when(sid == NS - 1)
    def _():
        pltpu.sync_copy(acc_sh.at[pl.ds((NS - 1) * ROWS_A, ROWS_LAST)],
                        out_hbm.at[cid, pl.ds((NS - 1) * ROWS_A, ROWS_LAST)])


@functools.lru_cache(maxsize=1)
def _get_segsum():
    mesh = plsc.VectorSubcoreMesh(
        core_axis_name="c", subcore_axis_name="s",
        num_cores=NC, num_subcores=NS)
    return pl.kernel(
        _segsum_body,
        out_type=jax.ShapeDtypeStruct((NC, N, D), _f32),
        mesh=mesh,
        compiler_params=pltpu.CompilerParams(use_tc_tiling_on_sc=False),
        scratch_types=[
            pltpu.VMEM((NCH, CHUNK), jnp.int32),
            pltpu.VMEM((NCH, CHUNK), jnp.int32),
            pltpu.VMEM((NBUF, CHUNK, D), _f32),
            pltpu.VMEM_SHARED((N, D), _f32),
            pltpu.SemaphoreType.DMA((NBUF,)),
            pltpu.SemaphoreType.DMA((NBUF,)),
            pltpu.SemaphoreType.DMA((2,)),
        ],
    )


def _segsum(vals, idx, zeros):
    return _get_segsum()(vals, idx, zeros)


def _code_body(acc_ref, r_ref, bl_ref, u_ref, y_ref):
    acc = acc_ref[0] + acc_ref[1]
    deg = acc[:, CODE:CODE + 1]
    invdeg = _f32(1.0) / jnp.maximum(deg, _f32(1.0))
    h = acc[:, :CODE] * invdeg + bl_ref[...] + r_ref[...]
    z = h + u_ref[...]
    z0 = z[:, :K]
    z1 = z[:, K:]
    e0 = jnp.exp(z0 - jnp.max(z0, axis=1, keepdims=True))
    e1 = jnp.exp(z1 - jnp.max(z1, axis=1, keepdims=True))
    y0 = e0 / jnp.sum(e0, axis=1, keepdims=True)
    y1 = e1 / jnp.sum(e1, axis=1, keepdims=True)
    pad = jnp.zeros((z.shape[0], D - CODE - 1), _f32)
    y_ref[...] = jnp.concatenate([y0, y1, invdeg, pad], axis=1)


def _code(acc1, r, bl1, u):
    blk = 1000
    return pl.pallas_call(
        _code_body,
        grid=(N // blk,),
        in_specs=[
            pl.BlockSpec((NC, blk, D), lambda i: (0, i, 0)),
            pl.BlockSpec((blk, CODE), lambda i: (i, 0)),
            pl.BlockSpec((1, CODE), lambda i: (0, 0)),
            pl.BlockSpec((blk, CODE), lambda i: (i, 0)),
        ],
        out_specs=pl.BlockSpec((blk, D), lambda i: (i, 0)),
        out_shape=jax.ShapeDtypeStruct((N, D), _f32),
    )(acc1, r, bl1, u)


def _dec_body(acc_ref, y_ref, wl_ref, wr_ref, bl_ref, out_ref):
    acc = acc_ref[0] + acc_ref[1]
    ypad = y_ref[...]
    invdeg = ypad[:, CODE:CODE + 1]
    mean2 = acc[:, :CODE] * invdeg
    out_ref[...] = (lax.dot_general(mean2, wl_ref[...], _DN,
                                    preferred_element_type=_f32)
                    + lax.dot_general(ypad[:, :CODE], wr_ref[...], _DN,
                                      preferred_element_type=_f32)
                    + bl_ref[...])


def _dec(acc2, y_pad, wl2, wr2, bl2):
    blk = 1000
    return pl.pallas_call(
        _dec_body,
        grid=(N // blk,),
        in_specs=[
            pl.BlockSpec((NC, blk, D), lambda i: (0, i, 0)),
            pl.BlockSpec((blk, D), lambda i: (i, 0)),
            pl.BlockSpec((IN_DIM, CODE), lambda i: (0, 0)),
            pl.BlockSpec((IN_DIM, CODE), lambda i: (0, 0)),
            pl.BlockSpec((1, IN_DIM), lambda i: (0, 0)),
        ],
        out_specs=pl.BlockSpec((blk, IN_DIM), lambda i: (i, 0)),
        out_shape=jax.ShapeDtypeStruct((N, IN_DIM), _f32),
    )(acc2, y_pad, wl2, wr2, bl2)


def kernel(x, edge_index, Wl1, bl1, Wr1, Wl2, bl2, Wr2):
    idx = _idx_prep(edge_index.astype(jnp.int32))
    idx = idx.reshape(2, NC * NS, NCH, CHUNK)

    p_pad, r = _proj(x, Wl1, Wr1)

    zeros = jnp.zeros((N, D), _f32)
    acc1 = _segsum(p_pad, idx, zeros)

    u = jax.random.uniform(jax.random.key(123), (N, 2, K),
                           dtype=_f32).reshape(N, CODE)
    g = _gum(u)
    y_pad = _code(acc1, r, bl1.reshape(1, CODE), g)

    acc2 = _segsum(y_pad, idx, zeros)

    return _dec(acc2, y_pad, Wl2, Wr2, bl2.reshape(1, IN_DIM))


# speedup vs baseline: 26.8182x; 1.0205x over previous
import functools

import jax
import jax.numpy as jnp
from jax import lax
from jax.experimental import pallas as pl
from jax.experimental.pallas import tpu as pltpu
from jax.experimental.pallas import tpu_sc as plsc

N = 10000
E = 320000
IN_DIM = 128
CODE = 20
K = 10
D = 32
EPS = 1e-20

NC = 2
NS = 16
E_PER_SC = E // NC
E_PER_TILE = E_PER_SC // NS
CHUNK = 80
NCH = E_PER_TILE // CHUNK
NBUF = 12
LOOKAHEAD = 8
ROWS_A = 624
ROWS_LAST = N - 15 * ROWS_A

_f32 = jnp.float32


_DN = (((1,), (1,)), ((), ()))


def _idx_body(e_ref, o_ref):
    e = e_ref[...]
    q = E // 4
    rows = [e[r // 4:r // 4 + 1, (r % 4) * q:(r % 4 + 1) * q]
            for r in range(8)]
    o_ref[...] = jnp.concatenate(rows, axis=0)


def _idx_prep(edge_index):
    return pl.pallas_call(
        _idx_body,
        out_shape=jax.ShapeDtypeStruct((8, E // 4), jnp.int32),
    )(edge_index)


def _gum_body(u_ref, g_ref):
    g_ref[...] = -jnp.log(-jnp.log(u_ref[...] + _f32(EPS)))


def _gum(u):
    blk = 2000
    return pl.pallas_call(
        _gum_body,
        grid=(N // blk,),
        in_specs=[pl.BlockSpec((blk, CODE), lambda i: (i, 0))],
        out_specs=pl.BlockSpec((blk, CODE), lambda i: (i, 0)),
        out_shape=jax.ShapeDtypeStruct((N, CODE), _f32),
    )(u)


def _proj_body(x_ref, wl_ref, wr_ref, p_ref, r_ref):
    xb = x_ref[...]
    p = lax.dot_general(xb, wl_ref[...], _DN, preferred_element_type=_f32)
    blk = p.shape[0]
    p_ref[...] = jnp.concatenate(
        [p, jnp.ones((blk, 1), _f32), jnp.zeros((blk, D - CODE - 1), _f32)],
        axis=1)
    r_ref[...] = lax.dot_general(xb, wr_ref[...], _DN,
                                 preferred_element_type=_f32)


def _proj(x, wl, wr):
    blk = 1000
    return pl.pallas_call(
        _proj_body,
        grid=(N // blk,),
        in_specs=[
            pl.BlockSpec((blk, IN_DIM), lambda i: (i, 0)),
            pl.BlockSpec((CODE, IN_DIM), lambda i: (0, 0)),
            pl.BlockSpec((CODE, IN_DIM), lambda i: (0, 0)),
        ],
        out_specs=[
            pl.BlockSpec((blk, D), lambda i: (i, 0)),
            pl.BlockSpec((blk, CODE), lambda i: (i, 0)),
        ],
        out_shape=[
            jax.ShapeDtypeStruct((N, D), _f32),
            jax.ShapeDtypeStruct((N, CODE), _f32),
        ],
    )(x, wl, wr)


def _segsum_body(vals_hbm, idx_hbm, zeros_hbm, out_hbm,
                 srcs_v, dsts_v, rows_v, acc_sh, gsem, ssem, isem):
    cid = lax.axis_index("c")
    sid = lax.axis_index("s")
    tid = cid * NS + sid
    row0 = pl.multiple_of(sid * ROWS_A, 8)

    e0 = tid * E_PER_TILE
    isrc = pltpu.async_copy(idx_hbm.at[0, pl.ds(e0, E_PER_TILE)], srcs_v,
                            isem.at[0])
    idst = pltpu.async_copy(idx_hbm.at[1, pl.ds(e0, E_PER_TILE)], dsts_v,
                            isem.at[1])
    isrc.wait()

    for k in range(LOOKAHEAD):
        pltpu.async_copy(vals_hbm.at[srcs_v.at[pl.ds(k * CHUNK, CHUNK)]],
                         rows_v.at[k], gsem.at[k])

    @pl.when(sid < NS - 1)
    def _():
        pltpu.sync_copy(zeros_hbm.at[pl.ds(row0, ROWS_A)],
                        acc_sh.at[pl.ds(row0, ROWS_A)])

    @pl.when(sid == NS - 1)
    def _():
        pltpu.sync_copy(zeros_hbm.at[pl.ds((NS - 1) * ROWS_A, ROWS_LAST)],
                        acc_sh.at[pl.ds((NS - 1) * ROWS_A, ROWS_LAST)])

    idst.wait()
    plsc.subcore_barrier()

    def _src(j):
        return srcs_v.at[pl.ds(j * CHUNK, CHUNK)]

    def _dst(j):
        return dsts_v.at[pl.ds(j * CHUNK, CHUNK)]

    def body(j, carry):
        b = lax.rem(j, NBUF)
        pltpu.make_async_copy(vals_hbm.at[_src(j)], rows_v.at[b],
                              gsem.at[b]).wait()
        pltpu.async_copy(rows_v.at[b], acc_sh.at[_dst(j)],
                         ssem.at[b], add=True)
        t = j + LOOKAHEAD
        tb = lax.rem(t, NBUF)

        @pl.when(jnp.logical_and(t >= NBUF, t < NCH))
        def _():
            pltpu.make_async_copy(rows_v.at[tb],
                                  acc_sh.at[_dst(t - NBUF)],
                                  ssem.at[tb]).wait()

        @pl.when(t < NCH)
        def _():
            pltpu.async_copy(vals_hbm.at[_src(t)], rows_v.at[tb],
                             gsem.at[tb])

        return carry

    lax.fori_loop(0, NCH, body, 0)
    for t in range(NCH - NBUF, NCH):
        pltpu.make_async_copy(rows_v.at[t % NBUF],
                              acc_sh.at[_dst(t)],
                              ssem.at[t % NBUF]).wait()
    plsc.subcore_barrier()

    @pl.when(sid < NS - 1)
    def _():
        pltpu.sync_copy(acc_sh.at[pl.ds(row0, ROWS_A)],
                        out_hbm.at[cid, pl.ds(row0, ROWS_A)])

    @pl.when(sid == NS - 1)
    def _():
        pltpu.sync_copy(acc_sh.at[pl.ds((NS - 1) * ROWS_A, ROWS_LAST)],
                        out_hbm.at[cid, pl.ds((NS - 1) * ROWS_A, ROWS_LAST)])


@functools.lru_cache(maxsize=1)
def _get_segsum():
    mesh = plsc.VectorSubcoreMesh(
        core_axis_name="c", subcore_axis_name="s",
        num_cores=NC, num_subcores=NS)
    return pl.kernel(
        _segsum_body,
        out_type=jax.ShapeDtypeStruct((NC, N, D), _f32),
        mesh=mesh,
        compiler_params=pltpu.CompilerParams(use_tc_tiling_on_sc=False),
        scratch_types=[
            pltpu.VMEM((E_PER_TILE,), jnp.int32),
            pltpu.VMEM((E_PER_TILE,), jnp.int32),
            pltpu.VMEM((NBUF, CHUNK, D), _f32),
            pltpu.VMEM_SHARED((N, D), _f32),
            pltpu.SemaphoreType.DMA((NBUF,)),
            pltpu.SemaphoreType.DMA((NBUF,)),
            pltpu.SemaphoreType.DMA((2,)),
        ],
    )


def _segsum(vals, idx, zeros):
    return _get_segsum()(vals, idx, zeros)


def _code_body(acc_ref, r_ref, bl_ref, u_ref, y_ref):
    acc = acc_ref[0] + acc_ref[1]
    deg = acc[:, CODE:CODE + 1]
    invdeg = _f32(1.0) / jnp.maximum(deg, _f32(1.0))
    h = acc[:, :CODE] * invdeg + bl_ref[...] + r_ref[...]
    z = h + u_ref[...]
    z0 = z[:, :K]
    z1 = z[:, K:]
    e0 = jnp.exp(z0 - jnp.max(z0, axis=1, keepdims=True))
    e1 = jnp.exp(z1 - jnp.max(z1, axis=1, keepdims=True))
    y0 = e0 / jnp.sum(e0, axis=1, keepdims=True)
    y1 = e1 / jnp.sum(e1, axis=1, keepdims=True)
    pad = jnp.zeros((z.shape[0], D - CODE - 1), _f32)
    y_ref[...] = jnp.concatenate([y0, y1, invdeg, pad], axis=1)


def _code(acc1, r, bl1, u):
    blk = 1000
    return pl.pallas_call(
        _code_body,
        grid=(N // blk,),
        in_specs=[
            pl.BlockSpec((NC, blk, D), lambda i: (0, i, 0)),
            pl.BlockSpec((blk, CODE), lambda i: (i, 0)),
            pl.BlockSpec((1, CODE), lambda i: (0, 0)),
            pl.BlockSpec((blk, CODE), lambda i: (i, 0)),
        ],
        out_specs=pl.BlockSpec((blk, D), lambda i: (i, 0)),
        out_shape=jax.ShapeDtypeStruct((N, D), _f32),
    )(acc1, r, bl1, u)


def _dec_body(acc_ref, y_ref, wl_ref, wr_ref, bl_ref, out_ref):
    acc = acc_ref[0] + acc_ref[1]
    ypad = y_ref[...]
    invdeg = ypad[:, CODE:CODE + 1]
    mean2 = acc[:, :CODE] * invdeg
    out_ref[...] = (lax.dot_general(mean2, wl_ref[...], _DN,
                                    preferred_element_type=_f32)
                    + lax.dot_general(ypad[:, :CODE], wr_ref[...], _DN,
                                      preferred_element_type=_f32)
                    + bl_ref[...])


def _dec(acc2, y_pad, wl2, wr2, bl2):
    blk = 1000
    return pl.pallas_call(
        _dec_body,
        grid=(N // blk,),
        in_specs=[
            pl.BlockSpec((NC, blk, D), lambda i: (0, i, 0)),
            pl.BlockSpec((blk, D), lambda i: (i, 0)),
            pl.BlockSpec((IN_DIM, CODE), lambda i: (0, 0)),
            pl.BlockSpec((IN_DIM, CODE), lambda i: (0, 0)),
            pl.BlockSpec((1, IN_DIM), lambda i: (0, 0)),
        ],
        out_specs=pl.BlockSpec((blk, IN_DIM), lambda i: (i, 0)),
        out_shape=jax.ShapeDtypeStruct((N, IN_DIM), _f32),
    )(acc2, y_pad, wl2, wr2, bl2)


def kernel(x, edge_index, Wl1, bl1, Wr1, Wl2, bl2, Wr2):
    idx = edge_index.astype(jnp.int32)

    p_pad, r = _proj(x, Wl1, Wr1)

    zeros = jnp.zeros((N, D), _f32)
    acc1 = _segsum(p_pad, idx, zeros)

    u = jax.random.uniform(jax.random.key(123), (N, 2, K),
                           dtype=_f32).reshape(N, CODE)
    g = _gum(u)
    y_pad = _code(acc1, r, bl1.reshape(1, CODE), g)

    acc2 = _segsum(y_pad, idx, zeros)

    return _dec(acc2, y_pad, Wl2, Wr2, bl2.reshape(1, IN_DIM))


# speedup vs baseline: 27.7436x; 1.0345x over previous
import functools

import jax
import jax.numpy as jnp
from jax import lax
from jax.experimental import pallas as pl
from jax.experimental.pallas import tpu as pltpu
from jax.experimental.pallas import tpu_sc as plsc

N = 10000
E = 320000
IN_DIM = 128
CODE = 20
K = 10
D = 32
EPS = 1e-20

NC = 2
NS = 16
E_PER_SC = E // NC
E_PER_TILE = E_PER_SC // NS
CHUNK = 80
NCH = E_PER_TILE // CHUNK
NBUF = 12
LOOKAHEAD = 8
ROWS_A = 624
ROWS_LAST = N - 15 * ROWS_A

_f32 = jnp.float32


_DN = (((1,), (1,)), ((), ()))


def _idx_body(e_ref, o_ref):
    e = e_ref[...]
    q = E // 4
    rows = [e[r // 4:r // 4 + 1, (r % 4) * q:(r % 4 + 1) * q]
            for r in range(8)]
    o_ref[...] = jnp.concatenate(rows, axis=0)


def _idx_prep(edge_index):
    return pl.pallas_call(
        _idx_body,
        out_shape=jax.ShapeDtypeStruct((8, E // 4), jnp.int32),
    )(edge_index)


def _gum_body(u_ref, g_ref):
    g_ref[...] = -jnp.log(-jnp.log(u_ref[...] + _f32(EPS)))


def _gum(u):
    blk = 2000
    return pl.pallas_call(
        _gum_body,
        grid=(N // blk,),
        in_specs=[pl.BlockSpec((blk, CODE), lambda i: (i, 0))],
        out_specs=pl.BlockSpec((blk, CODE), lambda i: (i, 0)),
        out_shape=jax.ShapeDtypeStruct((N, CODE), _f32),
    )(u)


def _proj_body(x_ref, wl_ref, wr_ref, p_ref, r_ref):
    xb = x_ref[...]
    p = lax.dot_general(xb, wl_ref[...], _DN, preferred_element_type=_f32)
    blk = p.shape[0]
    p_ref[...] = jnp.concatenate(
        [p, jnp.ones((blk, 1), _f32), jnp.zeros((blk, D - CODE - 1), _f32)],
        axis=1)
    r_ref[...] = lax.dot_general(xb, wr_ref[...], _DN,
                                 preferred_element_type=_f32)


def _proj(x, wl, wr):
    blk = 1000
    return pl.pallas_call(
        _proj_body,
        grid=(N // blk,),
        in_specs=[
            pl.BlockSpec((blk, IN_DIM), lambda i: (i, 0)),
            pl.BlockSpec((CODE, IN_DIM), lambda i: (0, 0)),
            pl.BlockSpec((CODE, IN_DIM), lambda i: (0, 0)),
        ],
        out_specs=[
            pl.BlockSpec((blk, D), lambda i: (i, 0)),
            pl.BlockSpec((blk, CODE), lambda i: (i, 0)),
        ],
        out_shape=[
            jax.ShapeDtypeStruct((N, D), _f32),
            jax.ShapeDtypeStruct((N, CODE), _f32),
        ],
    )(x, wl, wr)


def _segsum_body(vals_hbm, idx_hbm, zeros_hbm, out_hbm,
                 srcs_v, dsts_v, rows_v, acc_sh, gsem, ssem, isem):
    cid = lax.axis_index("c")
    sid = lax.axis_index("s")
    tid = cid * NS + sid
    row0 = pl.multiple_of(sid * ROWS_A, 8)

    e0 = tid * E_PER_TILE
    isrc = pltpu.async_copy(idx_hbm.at[0, pl.ds(e0, E_PER_TILE)], srcs_v,
                            isem.at[0])
    idst = pltpu.async_copy(idx_hbm.at[1, pl.ds(e0, E_PER_TILE)], dsts_v,
                            isem.at[1])
    isrc.wait()

    for k in range(LOOKAHEAD):
        pltpu.async_copy(vals_hbm.at[srcs_v.at[pl.ds(k * CHUNK, CHUNK)]],
                         rows_v.at[k], gsem.at[k])

    @pl.when(sid < NS - 1)
    def _():
        pltpu.sync_copy(zeros_hbm.at[pl.ds(row0, ROWS_A)],
                        acc_sh.at[pl.ds(row0, ROWS_A)])

    @pl.when(sid == NS - 1)
    def _():
        pltpu.sync_copy(zeros_hbm.at[pl.ds((NS - 1) * ROWS_A, ROWS_LAST)],
                        acc_sh.at[pl.ds((NS - 1) * ROWS_A, ROWS_LAST)])

    idst.wait()
    plsc.subcore_barrier()

    def _src(j):
        return srcs_v.at[pl.ds(j * CHUNK, CHUNK)]

    def _dst(j):
        return dsts_v.at[pl.ds(j * CHUNK, CHUNK)]

    def body(j, carry):
        b = lax.rem(j, NBUF)
        pltpu.make_async_copy(vals_hbm.at[_src(j)], rows_v.at[b],
                              gsem.at[b]).wait()
        pltpu.async_copy(rows_v.at[b], acc_sh.at[_dst(j)],
                         ssem.at[b], add=True)
        t = j + LOOKAHEAD
        tb = lax.rem(t, NBUF)

        @pl.when(jnp.logical_and(t >= NBUF, t < NCH))
        def _():
            pltpu.make_async_copy(rows_v.at[tb],
                                  acc_sh.at[_dst(t - NBUF)],
                                  ssem.at[tb]).wait()

        @pl.when(t < NCH)
        def _():
            pltpu.async_copy(vals_hbm.at[_src(t)], rows_v.at[tb],
                             gsem.at[tb])

        return carry

    lax.fori_loop(0, NCH, body, 0)
    for t in range(NCH - NBUF, NCH):
        pltpu.make_async_copy(rows_v.at[t % NBUF],
                              acc_sh.at[_dst(t)],
                              ssem.at[t % NBUF]).wait()
    plsc.subcore_barrier()

    @pl.when(sid < NS - 1)
    def _():
        pltpu.sync_copy(acc_sh.at[pl.ds(row0, ROWS_A)],
                        out_hbm.at[cid, pl.ds(row0, ROWS_A)])

    @pl.when(sid == NS - 1)
    def _():
        pltpu.sync_copy(acc_sh.at[pl.ds((NS - 1) * ROWS_A, ROWS_LAST)],
                        out_hbm.at[cid, pl.ds((NS - 1) * ROWS_A, ROWS_LAST)])


@functools.lru_cache(maxsize=1)
def _get_segsum():
    mesh = plsc.VectorSubcoreMesh(
        core_axis_name="c", subcore_axis_name="s",
        num_cores=NC, num_subcores=NS)
    return pl.kernel(
        _segsum_body,
        out_type=jax.ShapeDtypeStruct((NC, N, D), _f32),
        mesh=mesh,
        compiler_params=pltpu.CompilerParams(use_tc_tiling_on_sc=False),
        scratch_types=[
            pltpu.VMEM((E_PER_TILE,), jnp.int32),
            pltpu.VMEM((E_PER_TILE,), jnp.int32),
            pltpu.VMEM((NBUF, CHUNK, D), _f32),
            pltpu.VMEM_SHARED((N, D), _f32),
            pltpu.SemaphoreType.DMA((NBUF,)),
            pltpu.SemaphoreType.DMA((NBUF,)),
            pltpu.SemaphoreType.DMA((2,)),
        ],
    )


def _segsum(vals, idx, zeros):
    return _get_segsum()(vals, idx, zeros)


def _code_body(acc_ref, r_ref, bl_ref, u_ref, y_ref):
    acc = acc_ref[0] + acc_ref[1]
    deg = acc[:, CODE:CODE + 1]
    invdeg = _f32(1.0) / jnp.maximum(deg, _f32(1.0))
    h = acc[:, :CODE] * invdeg + bl_ref[...] + r_ref[...]
    z = h + u_ref[...]
    z0 = z[:, :K]
    z1 = z[:, K:]
    e0 = jnp.exp(z0 - jnp.max(z0, axis=1, keepdims=True))
    e1 = jnp.exp(z1 - jnp.max(z1, axis=1, keepdims=True))
    y0 = e0 / jnp.sum(e0, axis=1, keepdims=True)
    y1 = e1 / jnp.sum(e1, axis=1, keepdims=True)
    pad = jnp.zeros((z.shape[0], D - CODE - 1), _f32)
    y_ref[...] = jnp.concatenate([y0, y1, invdeg, pad], axis=1)


def _code(acc1, r, bl1, u):
    blk = 2000
    return pl.pallas_call(
        _code_body,
        grid=(N // blk,),
        in_specs=[
            pl.BlockSpec((NC, blk, D), lambda i: (0, i, 0)),
            pl.BlockSpec((blk, CODE), lambda i: (i, 0)),
            pl.BlockSpec((1, CODE), lambda i: (0, 0)),
            pl.BlockSpec((blk, CODE), lambda i: (i, 0)),
        ],
        out_specs=pl.BlockSpec((blk, D), lambda i: (i, 0)),
        out_shape=jax.ShapeDtypeStruct((N, D), _f32),
    )(acc1, r, bl1, u)


def _dec_body(acc_ref, y_ref, wl_ref, wr_ref, bl_ref, out_ref):
    acc = acc_ref[0] + acc_ref[1]
    ypad = y_ref[...]
    invdeg = ypad[:, CODE:CODE + 1]
    mean2 = acc[:, :CODE] * invdeg
    out_ref[...] = (lax.dot_general(mean2, wl_ref[...], _DN,
                                    preferred_element_type=_f32)
                    + lax.dot_general(ypad[:, :CODE], wr_ref[...], _DN,
                                      preferred_element_type=_f32)
                    + bl_ref[...])


def _dec(acc2, y_pad, wl2, wr2, bl2):
    blk = 2000
    return pl.pallas_call(
        _dec_body,
        grid=(N // blk,),
        in_specs=[
            pl.BlockSpec((NC, blk, D), lambda i: (0, i, 0)),
            pl.BlockSpec((blk, D), lambda i: (i, 0)),
            pl.BlockSpec((IN_DIM, CODE), lambda i: (0, 0)),
            pl.BlockSpec((IN_DIM, CODE), lambda i: (0, 0)),
            pl.BlockSpec((1, IN_DIM), lambda i: (0, 0)),
        ],
        out_specs=pl.BlockSpec((blk, IN_DIM), lambda i: (i, 0)),
        out_shape=jax.ShapeDtypeStruct((N, IN_DIM), _f32),
    )(acc2, y_pad, wl2, wr2, bl2)


def kernel(x, edge_index, Wl1, bl1, Wr1, Wl2, bl2, Wr2):
    idx = edge_index.astype(jnp.int32)

    p_pad, r = _proj(x, Wl1, Wr1)

    zeros = jnp.zeros((N, D), _f32)
    acc1 = _segsum(p_pad, idx, zeros)

    u = jax.random.uniform(jax.random.key(123), (N, 2, K),
                           dtype=_f32).reshape(N, CODE)
    g = _gum(u)
    y_pad = _code(acc1, r, bl1.reshape(1, CODE), g)

    acc2 = _segsum(y_pad, idx, zeros)

    return _dec(acc2, y_pad, Wl2, Wr2, bl2.reshape(1, IN_DIM))
